# Initial kernel scaffold; baseline (speedup 1.0000x reference)
import jax
import jax.numpy as jnp
from jax.experimental import pallas as pl


def kernel(x, edge_index, edge_attr, params):
    raise NotImplementedError("write your pallas kernel here")


# speedup vs baseline: 2.5651x; 2.5651x over previous
import jax
import jax.numpy as jnp
from jax import lax
from jax.experimental import pallas as pl
from jax.experimental.pallas import tpu as pltpu
from jax.experimental.pallas import tpu_sc as plsc

N_NODES = 10000
N_PAD = 10240
N_EDGES = 320000
E_PAD = 327680
NUM_FEATURES = 128
EDGE_DIM = 16
N_HIDDEN = 100
HP = 128

NS = 16
LANES = 16

EB = 80
NBLK = E_PAD // EB
NBLK_T = NBLK // NS
ROWS_T = N_PAD // NS
IGRP = 8
DUMMY = N_NODES

_f32 = jnp.float32


def _pad_w(w):
    return jnp.pad(w, ((0, HP - w.shape[0]), (0, HP - w.shape[1])))


def _pad_v(v):
    return jnp.pad(v, (0, HP - v.shape[0])).reshape(1, HP)


def _embed_nodes_body(x_ref, w_ref, b_ref, o_ref):
    o_ref[...] = (
        jnp.dot(x_ref[...], w_ref[...], preferred_element_type=_f32) + b_ref[...]
    )


def _embed_nodes(x, w, b):
    return pl.pallas_call(
        _embed_nodes_body,
        out_shape=jax.ShapeDtypeStruct((N_PAD, HP), _f32),
    )(x, w, b)


def _embed_edges_body(a_ref, w_ref, b_ref, o_ref):
    o_ref[...] = (
        jnp.dot(a_ref[...], w_ref[...], preferred_element_type=_f32) + b_ref[...]
    )


def _embed_edges(edge_attr, w, b):
    BE = 8192
    return pl.pallas_call(
        _embed_edges_body,
        grid=(E_PAD // BE,),
        in_specs=[
            pl.BlockSpec((BE, EDGE_DIM), lambda i: (i, 0)),
            pl.BlockSpec((EDGE_DIM, HP), lambda i: (0, 0)),
            pl.BlockSpec((1, HP), lambda i: (0, 0)),
        ],
        out_specs=pl.BlockSpec((BE, HP), lambda i: (i, 0)),
        out_shape=jax.ShapeDtypeStruct((E_PAD, HP), _f32),
    )(edge_attr, w, b)


def _layer_body(h_ref, of_ref, orr_ref,
                wf1, bf1, wf2, bf2, gf, betaf,
                wr1, br1, wr2, br2, gr, betar,
                o_ref):
    h = h_ref[...]

    def branch(o, w1, b1, w2, b2, g, beta):
        z = jnp.maximum(
            jnp.dot(o, w1[...], preferred_element_type=_f32) + b1[...], 0.0)
        z = jnp.dot(z, w2[...], preferred_element_type=_f32) + b2[...]
        z10 = z[:N_NODES]
        mu = jnp.mean(z10, axis=0, keepdims=True)
        var = jnp.mean((z10 - mu) * (z10 - mu), axis=0, keepdims=True)
        zn = (z - mu) * lax.rsqrt(var + 1e-5) * g[...] + beta[...]
        return jnp.maximum(zn, 0.0)

    hf = branch(of_ref[...], wf1, bf1, wf2, bf2, gf, betaf)
    hr = branch(orr_ref[...], wr1, br1, wr2, br2, gr, betar)
    o_ref[...] = (h + hf + hr) * (1.0 / 3.0)


def _tc_layer(h, out_f, out_r, wf, wr):
    return pl.pallas_call(
        _layer_body,
        out_shape=jax.ShapeDtypeStruct((N_PAD, HP), _f32),
    )(h, out_f, out_r, *wf, *wr)


def _head_body(h_ref, w1, b1, w2, b2, w3, b3, o_ref):
    hh = h_ref[...][:N_NODES]
    z = jnp.maximum(
        jnp.dot(hh, w1[...], preferred_element_type=_f32) + b1[...], 0.0)
    z = jnp.maximum(
        jnp.dot(z, w2[...], preferred_element_type=_f32) + b2[...], 0.0)
    o_ref[...] = jnp.dot(z, w3[...], preferred_element_type=_f32) + b3[...]


def _tc_head(h, w1, b1, w2, b2, w3, b3):
    return pl.pallas_call(
        _head_body,
        out_shape=jax.ShapeDtypeStruct((N_NODES, 2), _f32),
    )(h, w1, b1, w2, b2, w3, b3)


def _sc_conv_body(h_hbm, ea_hbm, src_hbm, dst_hbm, of_hbm, orr_hbm,
                  accum, idx_g, idx_s, rows, eab, sem):
    c = lax.axis_index("c")
    s = lax.axis_index("s")

    rsl = pl.ds(s * ROWS_T, ROWS_T)
    pltpu.sync_copy(h_hbm.at[rsl], accum.at[rsl])

    plsc.subcore_barrier()

    blk0 = s * NBLK_T

    def group(g, carry):
        gsl = pl.ds(blk0 + g * IGRP, IGRP)

        @pl.when(c == 0)
        def _():
            pltpu.sync_copy(src_hbm.at[gsl], idx_g)
            pltpu.sync_copy(dst_hbm.at[gsl], idx_s)

        @pl.when(c != 0)
        def _():
            pltpu.sync_copy(dst_hbm.at[gsl], idx_g)
            pltpu.sync_copy(src_hbm.at[gsl], idx_s)

        def step(k, carry1):
            gcp = pltpu.async_copy(h_hbm.at[idx_g.at[k]], rows, sem)
            pltpu.sync_copy(
                ea_hbm.at[pl.ds((blk0 + g * IGRP + k) * EB, EB)], eab)
            gcp.wait()

            def row_step(r4, carry2):
                for rr in range(4):
                    for j in range(HP // LANES):
                        sl = pl.ds(j * LANES, LANES)
                        r = r4 * 4 + rr
                        rows[r, sl] = jnp.maximum(rows[r, sl] + eab[r, sl], 0.0)
                return carry2

            lax.fori_loop(0, EB // 4, row_step, 0, unroll=False)
            pltpu.sync_copy(rows, accum.at[idx_s.at[k]], add=True)
            return carry1

        lax.fori_loop(0, IGRP, step, 0, unroll=False)
        return carry

    lax.fori_loop(0, NBLK_T // IGRP, group, 0, unroll=False)

    plsc.subcore_barrier()

    @pl.when(c == 0)
    def _():
        pltpu.sync_copy(accum.at[rsl], of_hbm.at[rsl])

    @pl.when(c != 0)
    def _():
        pltpu.sync_copy(accum.at[rsl], orr_hbm.at[rsl])


def _sc_conv(h, ea, src2d, dst2d):
    mesh = plsc.VectorSubcoreMesh(core_axis_name="c", subcore_axis_name="s")
    f = pl.kernel(
        _sc_conv_body,
        out_type=(
            jax.ShapeDtypeStruct((N_PAD, HP), _f32),
            jax.ShapeDtypeStruct((N_PAD, HP), _f32),
        ),
        mesh=mesh,
        scratch_types=[
            pltpu.VMEM_SHARED((N_PAD, HP), _f32),
            pltpu.VMEM((IGRP, EB), jnp.int32),
            pltpu.VMEM((IGRP, EB), jnp.int32),
            pltpu.VMEM((EB, HP), _f32),
            pltpu.VMEM((EB, HP), _f32),
            pltpu.SemaphoreType.DMA,
        ],
    )
    return f(h, ea, src2d, dst2d)


def kernel(x, edge_index, edge_attr, params):
    p = params
    epad = E_PAD - N_EDGES
    src2d = jnp.concatenate(
        [edge_index[0], jnp.full((epad,), DUMMY, jnp.int32)]).reshape(NBLK, EB)
    dst2d = jnp.concatenate(
        [edge_index[1], jnp.full((epad,), DUMMY, jnp.int32)]).reshape(NBLK, EB)
    x_pad = jnp.pad(x, ((0, N_PAD - N_NODES), (0, 0)))
    ea_pad = jnp.pad(edge_attr, ((0, epad), (0, 0)))

    node_w = jnp.pad(p['node_emb_w'], ((0, 0), (0, HP - N_HIDDEN)))
    node_b = _pad_v(p['node_emb_b'])
    edge_w = jnp.pad(p['edge_emb_w'], ((0, 0), (0, HP - N_HIDDEN)))
    edge_b = _pad_v(p['edge_emb_b'])

    h = _embed_nodes(x_pad, node_w, node_b)
    ea = _embed_edges(ea_pad, edge_w, edge_b)

    for i in range(2):
        out_f, out_r = _sc_conv(h, ea, src2d, dst2d)
        wf = (_pad_w(p[f'conv_f{i}_w1']), _pad_v(p[f'conv_f{i}_b1']),
              _pad_w(p[f'conv_f{i}_w2']), _pad_v(p[f'conv_f{i}_b2']),
              _pad_v(p[f'bn_f{i}_g']), _pad_v(p[f'bn_f{i}_b']))
        wr = (_pad_w(p[f'conv_r{i}_w1']), _pad_v(p[f'conv_r{i}_b1']),
              _pad_w(p[f'conv_r{i}_w2']), _pad_v(p[f'conv_r{i}_b2']),
              _pad_v(p[f'bn_r{i}_g']), _pad_v(p[f'bn_r{i}_b']))
        h = _tc_layer(h, out_f, out_r, wf, wr)

    w1 = jnp.pad(p['mlp_w1'], ((0, HP - N_HIDDEN), (0, 0)))
    out = _tc_head(h, w1, p['mlp_b1'].reshape(1, -1),
                   p['mlp_w2'], p['mlp_b2'].reshape(1, -1),
                   p['mlp_w3'], p['mlp_b3'].reshape(1, -1))
    return out
